# Initial kernel scaffold; baseline (speedup 1.0000x reference)
import jax
import jax.numpy as jnp
from jax.experimental import pallas as pl


def kernel(x, graph, edge_index):
    raise NotImplementedError("write your pallas kernel here")


# speedup vs baseline: 1.9927x; 1.9927x over previous
import jax
import jax.numpy as jnp
from jax.experimental import pallas as pl


_BLK_ROWS = 1000


def _softmax_maxpool_body(x_ref, cons_ref, obj_ref):
    i = pl.program_id(0)
    xb = x_ref[...]
    m = jnp.max(xb, axis=1, keepdims=True)
    e = jnp.exp(xb - m)
    s = jnp.sum(e, axis=1, keepdims=True)
    c = e / s
    cons_ref[...] = c
    pmax = jnp.max(c, axis=0, keepdims=True)

    @pl.when(i == 0)
    def _init():
        obj_ref[...] = pmax

    @pl.when(i > 0)
    def _acc():
        obj_ref[...] = jnp.maximum(obj_ref[...], pmax)


def kernel(x, graph, edge_index):
    del graph, edge_index
    n, d = x.shape
    blk = _BLK_ROWS if n % _BLK_ROWS == 0 else n
    grid = n // blk
    cons, obj = pl.pallas_call(
        _softmax_maxpool_body,
        grid=(grid,),
        in_specs=[pl.BlockSpec((blk, d), lambda i: (i, 0))],
        out_specs=(
            pl.BlockSpec((blk, d), lambda i: (i, 0)),
            pl.BlockSpec((1, d), lambda i: (0, 0)),
        ),
        out_shape=(
            jax.ShapeDtypeStruct((n, d), x.dtype),
            jax.ShapeDtypeStruct((1, d), x.dtype),
        ),
    )(x)
    return (cons, obj)


# speedup vs baseline: 2.4575x; 1.2332x over previous
import jax
import jax.numpy as jnp
from jax.experimental import pallas as pl


_BLK_ROWS = 2000


def _softmax_maxpool_body(x_ref, cons_ref, obj_ref):
    i = pl.program_id(0)
    xb = x_ref[...]
    m = jnp.max(xb, axis=1, keepdims=True)
    e = jnp.exp(xb - m)
    s = jnp.sum(e, axis=1, keepdims=True)
    c = e / s
    cons_ref[...] = c
    pmax = jnp.max(c, axis=0, keepdims=True)

    @pl.when(i == 0)
    def _init():
        obj_ref[...] = pmax

    @pl.when(i > 0)
    def _acc():
        obj_ref[...] = jnp.maximum(obj_ref[...], pmax)


def kernel(x, graph, edge_index):
    del graph, edge_index
    n, d = x.shape
    blk = _BLK_ROWS if n % _BLK_ROWS == 0 else n
    grid = n // blk
    cons, obj = pl.pallas_call(
        _softmax_maxpool_body,
        grid=(grid,),
        in_specs=[pl.BlockSpec((blk, d), lambda i: (i, 0))],
        out_specs=(
            pl.BlockSpec((blk, d), lambda i: (i, 0)),
            pl.BlockSpec((1, d), lambda i: (0, 0)),
        ),
        out_shape=(
            jax.ShapeDtypeStruct((n, d), x.dtype),
            jax.ShapeDtypeStruct((1, d), x.dtype),
        ),
    )(x)
    return (cons, obj)


# speedup vs baseline: 3.0804x; 1.2535x over previous
import jax
import jax.numpy as jnp
from jax.experimental import pallas as pl


_BLK_ROWS = 5000


def _softmax_maxpool_body(x_ref, cons_ref, obj_ref):
    i = pl.program_id(0)
    xb = x_ref[...]
    m = jnp.max(xb, axis=1, keepdims=True)
    e = jnp.exp(xb - m)
    s = jnp.sum(e, axis=1, keepdims=True)
    c = e / s
    cons_ref[...] = c
    pmax = jnp.max(c, axis=0, keepdims=True)

    @pl.when(i == 0)
    def _init():
        obj_ref[...] = pmax

    @pl.when(i > 0)
    def _acc():
        obj_ref[...] = jnp.maximum(obj_ref[...], pmax)


def kernel(x, graph, edge_index):
    del graph, edge_index
    n, d = x.shape
    blk = _BLK_ROWS if n % _BLK_ROWS == 0 else n
    grid = n // blk
    cons, obj = pl.pallas_call(
        _softmax_maxpool_body,
        grid=(grid,),
        in_specs=[pl.BlockSpec((blk, d), lambda i: (i, 0))],
        out_specs=(
            pl.BlockSpec((blk, d), lambda i: (i, 0)),
            pl.BlockSpec((1, d), lambda i: (0, 0)),
        ),
        out_shape=(
            jax.ShapeDtypeStruct((n, d), x.dtype),
            jax.ShapeDtypeStruct((1, d), x.dtype),
        ),
    )(x)
    return (cons, obj)
